# Initial kernel scaffold; baseline (speedup 1.0000x reference)
import jax
import jax.numpy as jnp
from jax.experimental import pallas as pl


def kernel(addr_number, addr_number_table, login_num_30d, login_num_30d_table, last7d_login_num, last7d_login_num_table, share_num_360d, share_num_360d_table, gmv_30d, gmv_30d_table, gmv_7d, gmv_7d_table, orders_30d, orders_30d_table, orders_7d, orders_7d_table, W0, b0, W1, b1, W2, b2):
    raise NotImplementedError("write your pallas kernel here")


# speedup vs baseline: 7.6923x; 7.6923x over previous
import numpy as np
import jax
import jax.numpy as jnp
from jax.experimental import pallas as pl

_B = 16384
_EMBED = 128
_NB = 21
_SCALES = (20.0, 2000.0, 2000.0, 2000.0, 2000.0, 1000.0, 40.0, 244.0)
_F = len(_SCALES)
_K = _F * _NB

def _build_consts():
    sel = np.zeros((_F, _K), np.float32)
    lo = np.full((1, _K), -np.inf, np.float32)
    hi = np.full((1, _K), np.inf, np.float32)
    for f, scale in enumerate(_SCALES):
        bnds = np.linspace(0.0, float(scale), 20).astype(np.float32)
        sel[f, f * _NB:(f + 1) * _NB] = 1.0
        lo[0, f * _NB + 1:(f + 1) * _NB] = bnds
        hi[0, f * _NB:(f + 1) * _NB - 1] = bnds
    return sel, lo, hi

_SEL, _LO, _HI = _build_consts()


def _prep_body(tables_ref, w0_ref, m_ref):
    for f in range(_F):
        m_ref[f] = jnp.dot(tables_ref[f], w0_ref[f],
                           preferred_element_type=jnp.float32, precision=jax.lax.Precision.HIGHEST)


def _fwd_body(vals_ref, sel_ref, lo_ref, hi_ref, m_ref, b0_ref,
              w1_ref, b1_ref, w2_ref, b2_ref, out_ref):
    x = jnp.dot(vals_ref[...], sel_ref[...],
                preferred_element_type=jnp.float32, precision=jax.lax.Precision.HIGHEST)
    oh = jnp.logical_and(x >= lo_ref[...], x < hi_ref[...]).astype(jnp.float32)
    h = jnp.dot(oh, m_ref[...], preferred_element_type=jnp.float32, precision=jax.lax.Precision.HIGHEST) + b0_ref[...]
    h = jnp.maximum(h, 0.0)
    h = jnp.dot(h, w1_ref[...], preferred_element_type=jnp.float32, precision=jax.lax.Precision.HIGHEST) + b1_ref[...]
    h = jnp.maximum(h, 0.0)
    out_ref[...] = (jnp.dot(h, w2_ref[...], preferred_element_type=jnp.float32, precision=jax.lax.Precision.HIGHEST)
                    + b2_ref[...])


def kernel(addr_number, addr_number_table, login_num_30d, login_num_30d_table,
           last7d_login_num, last7d_login_num_table, share_num_360d,
           share_num_360d_table, gmv_30d, gmv_30d_table, gmv_7d, gmv_7d_table,
           orders_30d, orders_30d_table, orders_7d, orders_7d_table,
           W0, b0, W1, b1, W2, b2):
    tables = jnp.stack([addr_number_table, login_num_30d_table,
                        last7d_login_num_table, share_num_360d_table,
                        gmv_30d_table, gmv_7d_table, orders_30d_table,
                        orders_7d_table])
    d1 = W0.shape[1]
    w0r = W0.reshape(_F, _EMBED, d1)
    m = pl.pallas_call(
        _prep_body,
        out_shape=jax.ShapeDtypeStruct((_F, _NB, d1), jnp.float32),
    )(tables, w0r)
    m = m.reshape(_K, d1)

    vals = jnp.stack([addr_number, login_num_30d, last7d_login_num,
                      share_num_360d, gmv_30d, gmv_7d, orders_30d, orders_7d],
                     axis=1)

    d2, d3 = W1.shape[1], W2.shape[1]
    tile = 2048
    grid = _B // tile
    out = pl.pallas_call(
        _fwd_body,
        grid=(grid,),
        in_specs=[
            pl.BlockSpec((tile, _F), lambda i: (i, 0)),
            pl.BlockSpec((_F, _K), lambda i: (0, 0)),
            pl.BlockSpec((1, _K), lambda i: (0, 0)),
            pl.BlockSpec((1, _K), lambda i: (0, 0)),
            pl.BlockSpec((_K, d1), lambda i: (0, 0)),
            pl.BlockSpec((1, d1), lambda i: (0, 0)),
            pl.BlockSpec((d1, d2), lambda i: (0, 0)),
            pl.BlockSpec((1, d2), lambda i: (0, 0)),
            pl.BlockSpec((d2, d3), lambda i: (0, 0)),
            pl.BlockSpec((1, d3), lambda i: (0, 0)),
        ],
        out_specs=pl.BlockSpec((tile, d3), lambda i: (i, 0)),
        out_shape=jax.ShapeDtypeStruct((_B, d3), jnp.float32),
    )(vals, jnp.asarray(_SEL), jnp.asarray(_LO), jnp.asarray(_HI), m,
      b0.reshape(1, d1), W1, b1.reshape(1, d2), W2, b2.reshape(1, d3))
    return out


# speedup vs baseline: 14.6925x; 1.9100x over previous
import numpy as np
import jax
import jax.numpy as jnp
from jax.experimental import pallas as pl
from jax.experimental.pallas import tpu as pltpu

_B = 16384
_EMBED = 128
_NB = 21
_SCALES = (20.0, 2000.0, 2000.0, 2000.0, 2000.0, 1000.0, 40.0, 244.0)
_F = len(_SCALES)
_K = _F * _NB

def _build_consts():
    sel = np.zeros((_F, _K), np.float32)
    lo = np.full((1, _K), -np.inf, np.float32)
    hi = np.full((1, _K), np.inf, np.float32)
    for f, scale in enumerate(_SCALES):
        bnds = np.linspace(0.0, float(scale), 20).astype(np.float32)
        sel[f, f * _NB:(f + 1) * _NB] = 1.0
        lo[0, f * _NB + 1:(f + 1) * _NB] = bnds
        hi[0, f * _NB:(f + 1) * _NB - 1] = bnds
    return sel, lo, hi

_SEL, _LO, _HI = _build_consts()


def _prep_body(tables_ref, w0_ref, m_ref):
    for f in range(_F):
        m_ref[f] = jnp.dot(tables_ref[f], w0_ref[f],
                           preferred_element_type=jnp.float32, precision=jax.lax.Precision.HIGHEST)


def _fwd_body(vals_ref, lo_ref, hi_ref, mhi_ref, mlo_ref, b0_ref,
              w1_ref, b1_ref, w2_ref, b2_ref, out_ref):
    pieces = []
    for f in range(_F):
        xf = vals_ref[:, f:f + 1]
        lof = lo_ref[:, f * _NB:(f + 1) * _NB]
        hif = hi_ref[:, f * _NB:(f + 1) * _NB]
        pieces.append(
            jnp.logical_and(xf >= lof, xf < hif).astype(jnp.bfloat16))
    oh = jnp.concatenate(pieces, axis=1)
    h = jnp.dot(oh, mhi_ref[...], preferred_element_type=jnp.float32)
    h = h + jnp.dot(oh, mlo_ref[...], preferred_element_type=jnp.float32)
    h = jnp.maximum(h + b0_ref[...], 0.0)
    hh = h.astype(jnp.bfloat16)
    hl = (h - hh.astype(jnp.float32)).astype(jnp.bfloat16)
    h = (jnp.dot(hh, w1_ref[...], preferred_element_type=jnp.float32)
         + jnp.dot(hl, w1_ref[...], preferred_element_type=jnp.float32))
    h = jnp.maximum(h + b1_ref[...], 0.0)
    hh = h.astype(jnp.bfloat16)
    hl = (h - hh.astype(jnp.float32)).astype(jnp.bfloat16)
    out_ref[...] = (jnp.dot(hh, w2_ref[...], preferred_element_type=jnp.float32)
                    + jnp.dot(hl, w2_ref[...], preferred_element_type=jnp.float32)
                    + b2_ref[...])


def kernel(addr_number, addr_number_table, login_num_30d, login_num_30d_table,
           last7d_login_num, last7d_login_num_table, share_num_360d,
           share_num_360d_table, gmv_30d, gmv_30d_table, gmv_7d, gmv_7d_table,
           orders_30d, orders_30d_table, orders_7d, orders_7d_table,
           W0, b0, W1, b1, W2, b2):
    tables = jnp.stack([addr_number_table, login_num_30d_table,
                        last7d_login_num_table, share_num_360d_table,
                        gmv_30d_table, gmv_7d_table, orders_30d_table,
                        orders_7d_table])
    d1 = W0.shape[1]
    w0r = W0.reshape(_F, _EMBED, d1)
    m = pl.pallas_call(
        _prep_body,
        out_shape=jax.ShapeDtypeStruct((_F, _NB, d1), jnp.float32),
    )(tables, w0r)
    m = m.reshape(_K, d1)
    m_hi = m.astype(jnp.bfloat16)
    m_lo = (m - m_hi.astype(jnp.float32)).astype(jnp.bfloat16)

    vals = jnp.stack([addr_number, login_num_30d, last7d_login_num,
                      share_num_360d, gmv_30d, gmv_7d, orders_30d, orders_7d],
                     axis=1)

    d2, d3 = W1.shape[1], W2.shape[1]
    tile = 2048
    grid = _B // tile
    out = pl.pallas_call(
        _fwd_body,
        grid=(grid,),
        in_specs=[
            pl.BlockSpec((tile, _F), lambda i: (i, 0)),
            pl.BlockSpec((1, _K), lambda i: (0, 0)),
            pl.BlockSpec((1, _K), lambda i: (0, 0)),
            pl.BlockSpec((_K, d1), lambda i: (0, 0)),
            pl.BlockSpec((_K, d1), lambda i: (0, 0)),
            pl.BlockSpec((1, d1), lambda i: (0, 0)),
            pl.BlockSpec((d1, d2), lambda i: (0, 0)),
            pl.BlockSpec((1, d2), lambda i: (0, 0)),
            pl.BlockSpec((d2, d3), lambda i: (0, 0)),
            pl.BlockSpec((1, d3), lambda i: (0, 0)),
        ],
        out_specs=pl.BlockSpec((tile, d3), lambda i: (i, 0)),
        out_shape=jax.ShapeDtypeStruct((_B, d3), jnp.float32),
        compiler_params=pltpu.CompilerParams(
            dimension_semantics=("parallel",)),
    )(vals, jnp.asarray(_LO), jnp.asarray(_HI), m_hi, m_lo,
      b0.reshape(1, d1), W1.astype(jnp.bfloat16), b1.reshape(1, d2),
      W2.astype(jnp.bfloat16), b2.reshape(1, d3))
    return out


# speedup vs baseline: 18.6845x; 1.2717x over previous
import numpy as np
import jax
import jax.numpy as jnp
from jax.experimental import pallas as pl
from jax.experimental.pallas import tpu as pltpu

_B = 16384
_EMBED = 128
_NB = 21
_SCALES = (20.0, 2000.0, 2000.0, 2000.0, 2000.0, 1000.0, 40.0, 244.0)
_F = len(_SCALES)
_K = _F * _NB

def _build_consts():
    sel = np.zeros((_F, _K), np.float32)
    lo = np.full((1, _K), -np.inf, np.float32)
    hi = np.full((1, _K), np.inf, np.float32)
    for f, scale in enumerate(_SCALES):
        bnds = np.linspace(0.0, float(scale), 20).astype(np.float32)
        sel[f, f * _NB:(f + 1) * _NB] = 1.0
        lo[0, f * _NB + 1:(f + 1) * _NB] = bnds
        hi[0, f * _NB:(f + 1) * _NB - 1] = bnds
    return sel, lo, hi

_SEL, _LO, _HI = _build_consts()


def _prep_body(tables_ref, w0_ref, m_ref):
    for f in range(_F):
        m_ref[f] = jnp.dot(tables_ref[f], w0_ref[f],
                           preferred_element_type=jnp.float32, precision=jax.lax.Precision.HIGHEST)


def _fwd_body(x_ref, lo_ref, hi_ref, mhi_ref, mlo_ref, b0_ref,
              w1_ref, b1_ref, w2_ref, b2_ref, out_ref):
    x = x_ref[...]
    oh = jnp.logical_and(x >= lo_ref[...], x < hi_ref[...]).astype(jnp.bfloat16)
    h = jnp.dot(oh, mhi_ref[...], preferred_element_type=jnp.float32)
    h = h + jnp.dot(oh, mlo_ref[...], preferred_element_type=jnp.float32)
    h = jnp.maximum(h + b0_ref[...], 0.0)
    h = jnp.dot(h.astype(jnp.bfloat16), w1_ref[...],
                preferred_element_type=jnp.float32)
    h = jnp.maximum(h + b1_ref[...], 0.0)
    out_ref[...] = (jnp.dot(h.astype(jnp.bfloat16), w2_ref[...],
                            preferred_element_type=jnp.float32)
                    + b2_ref[...])


def kernel(addr_number, addr_number_table, login_num_30d, login_num_30d_table,
           last7d_login_num, last7d_login_num_table, share_num_360d,
           share_num_360d_table, gmv_30d, gmv_30d_table, gmv_7d, gmv_7d_table,
           orders_30d, orders_30d_table, orders_7d, orders_7d_table,
           W0, b0, W1, b1, W2, b2):
    tables = jnp.stack([addr_number_table, login_num_30d_table,
                        last7d_login_num_table, share_num_360d_table,
                        gmv_30d_table, gmv_7d_table, orders_30d_table,
                        orders_7d_table])
    d1 = W0.shape[1]
    w0r = W0.reshape(_F, _EMBED, d1)
    m = pl.pallas_call(
        _prep_body,
        out_shape=jax.ShapeDtypeStruct((_F, _NB, d1), jnp.float32),
    )(tables, w0r)
    m = m.reshape(_K, d1)
    m_hi = m.astype(jnp.bfloat16)
    m_lo = (m - m_hi.astype(jnp.float32)).astype(jnp.bfloat16)

    vals = jnp.stack([addr_number, login_num_30d, last7d_login_num,
                      share_num_360d, gmv_30d, gmv_7d, orders_30d, orders_7d],
                     axis=1)
    x_exp = jnp.repeat(vals, _NB, axis=1)

    d2, d3 = W1.shape[1], W2.shape[1]
    tile = 2048
    grid = _B // tile
    out = pl.pallas_call(
        _fwd_body,
        grid=(grid,),
        in_specs=[
            pl.BlockSpec((tile, _K), lambda i: (i, 0)),
            pl.BlockSpec((1, _K), lambda i: (0, 0)),
            pl.BlockSpec((1, _K), lambda i: (0, 0)),
            pl.BlockSpec((_K, d1), lambda i: (0, 0)),
            pl.BlockSpec((_K, d1), lambda i: (0, 0)),
            pl.BlockSpec((1, d1), lambda i: (0, 0)),
            pl.BlockSpec((d1, d2), lambda i: (0, 0)),
            pl.BlockSpec((1, d2), lambda i: (0, 0)),
            pl.BlockSpec((d2, d3), lambda i: (0, 0)),
            pl.BlockSpec((1, d3), lambda i: (0, 0)),
        ],
        out_specs=pl.BlockSpec((tile, d3), lambda i: (i, 0)),
        out_shape=jax.ShapeDtypeStruct((_B, d3), jnp.float32),
        compiler_params=pltpu.CompilerParams(
            dimension_semantics=("parallel",)),
    )(x_exp, jnp.asarray(_LO), jnp.asarray(_HI), m_hi, m_lo,
      b0.reshape(1, d1), W1.astype(jnp.bfloat16), b1.reshape(1, d2),
      W2.astype(jnp.bfloat16), b2.reshape(1, d3))
    return out


# speedup vs baseline: 20.0079x; 1.0708x over previous
import numpy as np
import jax
import jax.numpy as jnp
from jax.experimental import pallas as pl
from jax.experimental.pallas import tpu as pltpu

_B = 16384
_EMBED = 128
_NB = 21
_SCALES = (20.0, 2000.0, 2000.0, 2000.0, 2000.0, 1000.0, 40.0, 244.0)
_F = len(_SCALES)
_K = _F * _NB

def _build_consts():
    sel = np.zeros((_F, _K), np.float32)
    lo = np.full((1, _K), -np.inf, np.float32)
    hi = np.full((1, _K), np.inf, np.float32)
    for f, scale in enumerate(_SCALES):
        bnds = np.linspace(0.0, float(scale), 20).astype(np.float32)
        sel[f, f * _NB:(f + 1) * _NB] = 1.0
        lo[0, f * _NB + 1:(f + 1) * _NB] = bnds
        hi[0, f * _NB:(f + 1) * _NB - 1] = bnds
    return sel, lo, hi

_SEL, _LO, _HI = _build_consts()


def _prep_body(tables_ref, w0_ref, m_ref):
    for f in range(_F):
        m_ref[f] = jnp.dot(tables_ref[f], w0_ref[f],
                           preferred_element_type=jnp.float32, precision=jax.lax.Precision.HIGHEST)


def _fwd_body(vals_ref, sel_ref, lo_ref, hi_ref, mhi_ref, mlo_ref, b0_ref,
              w1_ref, b1_ref, w2_ref, b2_ref, out_ref):
    v = vals_ref[...]
    v1 = v.astype(jnp.bfloat16)
    r = v - v1.astype(jnp.float32)
    v2 = r.astype(jnp.bfloat16)
    v3 = (r - v2.astype(jnp.float32)).astype(jnp.bfloat16)
    x = jnp.dot(v1, sel_ref[...], preferred_element_type=jnp.float32)
    x = x + jnp.dot(v2, sel_ref[...], preferred_element_type=jnp.float32)
    x = x + jnp.dot(v3, sel_ref[...], preferred_element_type=jnp.float32)
    oh = jnp.logical_and(x >= lo_ref[...], x < hi_ref[...]).astype(jnp.bfloat16)
    h = jnp.dot(oh, mhi_ref[...], preferred_element_type=jnp.float32)
    h = h + jnp.dot(oh, mlo_ref[...], preferred_element_type=jnp.float32)
    h = jnp.maximum(h + b0_ref[...], 0.0)
    h = jnp.dot(h.astype(jnp.bfloat16), w1_ref[...],
                preferred_element_type=jnp.float32)
    h = jnp.maximum(h + b1_ref[...], 0.0)
    out_ref[...] = (jnp.dot(h.astype(jnp.bfloat16), w2_ref[...],
                            preferred_element_type=jnp.float32)
                    + b2_ref[...])


def kernel(addr_number, addr_number_table, login_num_30d, login_num_30d_table,
           last7d_login_num, last7d_login_num_table, share_num_360d,
           share_num_360d_table, gmv_30d, gmv_30d_table, gmv_7d, gmv_7d_table,
           orders_30d, orders_30d_table, orders_7d, orders_7d_table,
           W0, b0, W1, b1, W2, b2):
    tables = jnp.stack([addr_number_table, login_num_30d_table,
                        last7d_login_num_table, share_num_360d_table,
                        gmv_30d_table, gmv_7d_table, orders_30d_table,
                        orders_7d_table])
    d1 = W0.shape[1]
    w0r = W0.reshape(_F, _EMBED, d1)
    m = pl.pallas_call(
        _prep_body,
        out_shape=jax.ShapeDtypeStruct((_F, _NB, d1), jnp.float32),
    )(tables, w0r)
    m = m.reshape(_K, d1)
    m_hi = m.astype(jnp.bfloat16)
    m_lo = (m - m_hi.astype(jnp.float32)).astype(jnp.bfloat16)

    vals = jnp.stack([addr_number, login_num_30d, last7d_login_num,
                      share_num_360d, gmv_30d, gmv_7d, orders_30d, orders_7d],
                     axis=1)

    d2, d3 = W1.shape[1], W2.shape[1]
    tile = 2048
    grid = _B // tile
    out = pl.pallas_call(
        _fwd_body,
        grid=(grid,),
        in_specs=[
            pl.BlockSpec((tile, _F), lambda i: (i, 0)),
            pl.BlockSpec((_F, _K), lambda i: (0, 0)),
            pl.BlockSpec((1, _K), lambda i: (0, 0)),
            pl.BlockSpec((1, _K), lambda i: (0, 0)),
            pl.BlockSpec((_K, d1), lambda i: (0, 0)),
            pl.BlockSpec((_K, d1), lambda i: (0, 0)),
            pl.BlockSpec((1, d1), lambda i: (0, 0)),
            pl.BlockSpec((d1, d2), lambda i: (0, 0)),
            pl.BlockSpec((1, d2), lambda i: (0, 0)),
            pl.BlockSpec((d2, d3), lambda i: (0, 0)),
            pl.BlockSpec((1, d3), lambda i: (0, 0)),
        ],
        out_specs=pl.BlockSpec((tile, d3), lambda i: (i, 0)),
        out_shape=jax.ShapeDtypeStruct((_B, d3), jnp.float32),
        compiler_params=pltpu.CompilerParams(
            dimension_semantics=("parallel",)),
    )(vals, jnp.asarray(_SEL).astype(jnp.bfloat16),
      jnp.asarray(_LO), jnp.asarray(_HI), m_hi, m_lo,
      b0.reshape(1, d1), W1.astype(jnp.bfloat16), b1.reshape(1, d2),
      W2.astype(jnp.bfloat16), b2.reshape(1, d3))
    return out


# speedup vs baseline: 20.1871x; 1.0090x over previous
import numpy as np
import jax
import jax.numpy as jnp
from jax.experimental import pallas as pl
from jax.experimental.pallas import tpu as pltpu

_B = 16384
_EMBED = 128
_NB = 21
_SCALES = (20.0, 2000.0, 2000.0, 2000.0, 2000.0, 1000.0, 40.0, 244.0)
_F = len(_SCALES)
_K = _F * _NB

def _build_consts():
    sel = np.zeros((_F, _K), np.float32)
    lo = np.full((1, _K), -np.inf, np.float32)
    hi = np.full((1, _K), np.inf, np.float32)
    for f, scale in enumerate(_SCALES):
        bnds = np.linspace(0.0, float(scale), 20).astype(np.float32)
        sel[f, f * _NB:(f + 1) * _NB] = 1.0
        lo[0, f * _NB + 1:(f + 1) * _NB] = bnds
        hi[0, f * _NB:(f + 1) * _NB - 1] = bnds
    return sel, lo, hi

_SEL, _LO, _HI = _build_consts()


def _prep_body(tables_ref, w0_ref, m_ref):
    for f in range(_F):
        m_ref[f] = jnp.dot(tables_ref[f], w0_ref[f],
                           preferred_element_type=jnp.float32, precision=jax.lax.Precision.HIGHEST)


def _fwd_body(vals_ref, sel_ref, lo_ref, hi_ref, mhi_ref, mlo_ref, b0_ref,
              w1_ref, b1_ref, w2_ref, b2_ref, out_ref):
    v = vals_ref[...]
    v1 = v.astype(jnp.bfloat16)
    r = v - v1.astype(jnp.float32)
    v2 = r.astype(jnp.bfloat16)
    v3 = (r - v2.astype(jnp.float32)).astype(jnp.bfloat16)
    x = jnp.dot(v1, sel_ref[...], preferred_element_type=jnp.float32)
    x = x + jnp.dot(v2, sel_ref[...], preferred_element_type=jnp.float32)
    x = x + jnp.dot(v3, sel_ref[...], preferred_element_type=jnp.float32)
    oh = jnp.logical_and(x >= lo_ref[...], x < hi_ref[...]).astype(jnp.bfloat16)
    h = jnp.dot(oh, mhi_ref[...], preferred_element_type=jnp.float32)
    h = h + jnp.dot(oh, mlo_ref[...], preferred_element_type=jnp.float32)
    h = jnp.maximum(h + b0_ref[...], 0.0)
    h = jnp.dot(h.astype(jnp.bfloat16), w1_ref[...],
                preferred_element_type=jnp.float32)
    h = jnp.maximum(h + b1_ref[...], 0.0)
    out_ref[...] = (jnp.dot(h.astype(jnp.bfloat16), w2_ref[...],
                            preferred_element_type=jnp.float32)
                    + b2_ref[...])


def kernel(addr_number, addr_number_table, login_num_30d, login_num_30d_table,
           last7d_login_num, last7d_login_num_table, share_num_360d,
           share_num_360d_table, gmv_30d, gmv_30d_table, gmv_7d, gmv_7d_table,
           orders_30d, orders_30d_table, orders_7d, orders_7d_table,
           W0, b0, W1, b1, W2, b2):
    tables = jnp.stack([addr_number_table, login_num_30d_table,
                        last7d_login_num_table, share_num_360d_table,
                        gmv_30d_table, gmv_7d_table, orders_30d_table,
                        orders_7d_table])
    d1 = W0.shape[1]
    w0r = W0.reshape(_F, _EMBED, d1)
    m = pl.pallas_call(
        _prep_body,
        out_shape=jax.ShapeDtypeStruct((_F, _NB, d1), jnp.float32),
    )(tables, w0r)
    m = m.reshape(_K, d1)
    m_hi = m.astype(jnp.bfloat16)
    m_lo = (m - m_hi.astype(jnp.float32)).astype(jnp.bfloat16)

    vals = jnp.stack([addr_number, login_num_30d, last7d_login_num,
                      share_num_360d, gmv_30d, gmv_7d, orders_30d, orders_7d],
                     axis=1)

    d2, d3 = W1.shape[1], W2.shape[1]
    tile = 4096
    grid = _B // tile
    out = pl.pallas_call(
        _fwd_body,
        grid=(grid,),
        in_specs=[
            pl.BlockSpec((tile, _F), lambda i: (i, 0)),
            pl.BlockSpec((_F, _K), lambda i: (0, 0)),
            pl.BlockSpec((1, _K), lambda i: (0, 0)),
            pl.BlockSpec((1, _K), lambda i: (0, 0)),
            pl.BlockSpec((_K, d1), lambda i: (0, 0)),
            pl.BlockSpec((_K, d1), lambda i: (0, 0)),
            pl.BlockSpec((1, d1), lambda i: (0, 0)),
            pl.BlockSpec((d1, d2), lambda i: (0, 0)),
            pl.BlockSpec((1, d2), lambda i: (0, 0)),
            pl.BlockSpec((d2, d3), lambda i: (0, 0)),
            pl.BlockSpec((1, d3), lambda i: (0, 0)),
        ],
        out_specs=pl.BlockSpec((tile, d3), lambda i: (i, 0)),
        out_shape=jax.ShapeDtypeStruct((_B, d3), jnp.float32),
        compiler_params=pltpu.CompilerParams(
            dimension_semantics=("parallel",)),
    )(vals, jnp.asarray(_SEL).astype(jnp.bfloat16),
      jnp.asarray(_LO), jnp.asarray(_HI), m_hi, m_lo,
      b0.reshape(1, d1), W1.astype(jnp.bfloat16), b1.reshape(1, d2),
      W2.astype(jnp.bfloat16), b2.reshape(1, d3))
    return out


# speedup vs baseline: 22.9653x; 1.1376x over previous
import numpy as np
import jax
import jax.numpy as jnp
from jax.experimental import pallas as pl
from jax.experimental.pallas import tpu as pltpu

_B = 16384
_EMBED = 128
_NB = 21
_SCALES = (20.0, 2000.0, 2000.0, 2000.0, 2000.0, 1000.0, 40.0, 244.0)
_F = len(_SCALES)
_K = _F * _NB
_TILE = 4096


def _build_consts():
    sel = np.zeros((_F, _K), np.float32)
    lo = np.full((1, _K), -np.inf, np.float32)
    hi = np.full((1, _K), np.inf, np.float32)
    for f, scale in enumerate(_SCALES):
        bnds = np.linspace(0.0, float(scale), 20).astype(np.float32)
        sel[f, f * _NB:(f + 1) * _NB] = 1.0
        lo[0, f * _NB + 1:(f + 1) * _NB] = bnds
        hi[0, f * _NB:(f + 1) * _NB - 1] = bnds
    return sel, lo, hi


_SEL, _LO, _HI = _build_consts()


def _body(vals_ref, sel_ref, lo_ref, hi_ref, tables_ref, w0_ref, b0_ref,
          w1_ref, b1_ref, w2_ref, b2_ref, out_ref,
          mhi_ref, mlo_ref, w1b_ref, w2b_ref):
    @pl.when(pl.program_id(0) == 0)
    def _init():
        for f in range(_F):
            mf = jnp.dot(tables_ref[f], w0_ref[f],
                         preferred_element_type=jnp.float32,
                         precision=jax.lax.Precision.HIGHEST)
            mh = mf.astype(jnp.bfloat16)
            mhi_ref[f * _NB:(f + 1) * _NB, :] = mh
            mlo_ref[f * _NB:(f + 1) * _NB, :] = (
                mf - mh.astype(jnp.float32)).astype(jnp.bfloat16)
        w1b_ref[...] = w1_ref[...].astype(jnp.bfloat16)
        w2b_ref[...] = w2_ref[...].astype(jnp.bfloat16)

    v = vals_ref[...]
    v1 = v.astype(jnp.bfloat16)
    r = v - v1.astype(jnp.float32)
    v2 = r.astype(jnp.bfloat16)
    v3 = (r - v2.astype(jnp.float32)).astype(jnp.bfloat16)
    x = jnp.dot(v1, sel_ref[...], preferred_element_type=jnp.float32)
    x = x + jnp.dot(v2, sel_ref[...], preferred_element_type=jnp.float32)
    x = x + jnp.dot(v3, sel_ref[...], preferred_element_type=jnp.float32)
    oh = jnp.logical_and(x >= lo_ref[...], x < hi_ref[...]).astype(jnp.bfloat16)
    h = jnp.dot(oh, mhi_ref[...], preferred_element_type=jnp.float32)
    h = h + jnp.dot(oh, mlo_ref[...], preferred_element_type=jnp.float32)
    h = jnp.maximum(h + b0_ref[...], 0.0)
    h = jnp.dot(h.astype(jnp.bfloat16), w1b_ref[...],
                preferred_element_type=jnp.float32)
    h = jnp.maximum(h + b1_ref[...], 0.0)
    out_ref[...] = (jnp.dot(h.astype(jnp.bfloat16), w2b_ref[...],
                            preferred_element_type=jnp.float32)
                    + b2_ref[...])


def kernel(addr_number, addr_number_table, login_num_30d, login_num_30d_table,
           last7d_login_num, last7d_login_num_table, share_num_360d,
           share_num_360d_table, gmv_30d, gmv_30d_table, gmv_7d, gmv_7d_table,
           orders_30d, orders_30d_table, orders_7d, orders_7d_table,
           W0, b0, W1, b1, W2, b2):
    tables = jnp.stack([addr_number_table, login_num_30d_table,
                        last7d_login_num_table, share_num_360d_table,
                        gmv_30d_table, gmv_7d_table, orders_30d_table,
                        orders_7d_table])
    d1 = W0.shape[1]
    w0r = W0.reshape(_F, _EMBED, d1)
    vals = jnp.stack([addr_number, login_num_30d, last7d_login_num,
                      share_num_360d, gmv_30d, gmv_7d, orders_30d, orders_7d],
                     axis=1)

    d2, d3 = W1.shape[1], W2.shape[1]
    grid = _B // _TILE
    out = pl.pallas_call(
        _body,
        grid=(grid,),
        in_specs=[
            pl.BlockSpec((_TILE, _F), lambda i: (i, 0)),
            pl.BlockSpec((_F, _K), lambda i: (0, 0)),
            pl.BlockSpec((1, _K), lambda i: (0, 0)),
            pl.BlockSpec((1, _K), lambda i: (0, 0)),
            pl.BlockSpec((_F, _NB, _EMBED), lambda i: (0, 0, 0)),
            pl.BlockSpec((_F, _EMBED, d1), lambda i: (0, 0, 0)),
            pl.BlockSpec((1, d1), lambda i: (0, 0)),
            pl.BlockSpec((d1, d2), lambda i: (0, 0)),
            pl.BlockSpec((1, d2), lambda i: (0, 0)),
            pl.BlockSpec((d2, d3), lambda i: (0, 0)),
            pl.BlockSpec((1, d3), lambda i: (0, 0)),
        ],
        out_specs=pl.BlockSpec((_TILE, d3), lambda i: (i, 0)),
        out_shape=jax.ShapeDtypeStruct((_B, d3), jnp.float32),
        scratch_shapes=[
            pltpu.VMEM((_K, d1), jnp.bfloat16),
            pltpu.VMEM((_K, d1), jnp.bfloat16),
            pltpu.VMEM((d1, d2), jnp.bfloat16),
            pltpu.VMEM((d2, d3), jnp.bfloat16),
        ],
    )(vals, jnp.asarray(_SEL).astype(jnp.bfloat16),
      jnp.asarray(_LO), jnp.asarray(_HI), tables, w0r,
      b0.reshape(1, d1), W1, b1.reshape(1, d2), W2, b2.reshape(1, d3))
    return out


# speedup vs baseline: 30.2299x; 1.3163x over previous
import numpy as np
import jax
import jax.numpy as jnp
from jax.experimental import pallas as pl
from jax.experimental.pallas import tpu as pltpu

_B = 16384
_EMBED = 128
_NB = 21
_SCALES = (20.0, 2000.0, 2000.0, 2000.0, 2000.0, 1000.0, 40.0, 244.0)
_F = len(_SCALES)
_K = _F * _NB
_TILE = 4096


def _build_consts():
    sel = np.zeros((_F, _K), np.float32)
    lo = np.full((1, _K), -np.inf, np.float32)
    hi = np.full((1, _K), np.inf, np.float32)
    for f, scale in enumerate(_SCALES):
        bnds = np.linspace(0.0, float(scale), 20).astype(np.float32)
        sel[f, f * _NB:(f + 1) * _NB] = 1.0
        lo[0, f * _NB + 1:(f + 1) * _NB] = bnds
        hi[0, f * _NB:(f + 1) * _NB - 1] = bnds
    return sel, lo, hi


_SEL, _LO, _HI = _build_consts()


def _body(vals_ref, sel_ref, lo_ref, hi_ref, tables_ref, w0_ref, b0_ref,
          w1_ref, b1_ref, w2_ref, b2_ref, out_ref,
          mhi_ref, mlo_ref, w1b_ref, w2b_ref):
    @pl.when(pl.program_id(0) == 0)
    def _init():
        for f in range(_F):
            mf = jnp.dot(tables_ref[f], w0_ref[f],
                         preferred_element_type=jnp.float32,
                         precision=jax.lax.Precision.HIGHEST)
            mh = mf.astype(jnp.bfloat16)
            mhi_ref[f * _NB:(f + 1) * _NB, :] = mh
            mlo_ref[f * _NB:(f + 1) * _NB, :] = (
                mf - mh.astype(jnp.float32)).astype(jnp.bfloat16)
        w1b_ref[...] = w1_ref[...].astype(jnp.bfloat16)
        w2b_ref[...] = w2_ref[...].astype(jnp.bfloat16)

    v = vals_ref[...]
    v1 = v.astype(jnp.bfloat16)
    r = v - v1.astype(jnp.float32)
    v2 = r.astype(jnp.bfloat16)
    v3 = (r - v2.astype(jnp.float32)).astype(jnp.bfloat16)
    v123 = jnp.concatenate([v1, v2, v3], axis=1)
    x = jnp.dot(v123, sel_ref[...], preferred_element_type=jnp.float32)
    oh = jnp.logical_and(x >= lo_ref[...], x < hi_ref[...]).astype(jnp.bfloat16)
    h = jnp.dot(oh, mhi_ref[...], preferred_element_type=jnp.float32)
    h = h + jnp.dot(oh, mlo_ref[...], preferred_element_type=jnp.float32)
    h = jnp.maximum(h + b0_ref[...], 0.0)
    h = jnp.dot(h.astype(jnp.bfloat16), w1b_ref[...],
                preferred_element_type=jnp.float32)
    h = jnp.maximum(h + b1_ref[...], 0.0)
    out_ref[...] = (jnp.dot(h.astype(jnp.bfloat16), w2b_ref[...],
                            preferred_element_type=jnp.float32)
                    + b2_ref[...])


def kernel(addr_number, addr_number_table, login_num_30d, login_num_30d_table,
           last7d_login_num, last7d_login_num_table, share_num_360d,
           share_num_360d_table, gmv_30d, gmv_30d_table, gmv_7d, gmv_7d_table,
           orders_30d, orders_30d_table, orders_7d, orders_7d_table,
           W0, b0, W1, b1, W2, b2):
    tables = jnp.stack([addr_number_table, login_num_30d_table,
                        last7d_login_num_table, share_num_360d_table,
                        gmv_30d_table, gmv_7d_table, orders_30d_table,
                        orders_7d_table])
    d1 = W0.shape[1]
    w0r = W0.reshape(_F, _EMBED, d1)
    vals = jnp.stack([addr_number, login_num_30d, last7d_login_num,
                      share_num_360d, gmv_30d, gmv_7d, orders_30d, orders_7d],
                     axis=1)

    d2, d3 = W1.shape[1], W2.shape[1]
    grid = _B // _TILE
    out = pl.pallas_call(
        _body,
        grid=(grid,),
        in_specs=[
            pl.BlockSpec((_TILE, _F), lambda i: (i, 0)),
            pl.BlockSpec((3 * _F, _K), lambda i: (0, 0)),
            pl.BlockSpec((1, _K), lambda i: (0, 0)),
            pl.BlockSpec((1, _K), lambda i: (0, 0)),
            pl.BlockSpec((_F, _NB, _EMBED), lambda i: (0, 0, 0)),
            pl.BlockSpec((_F, _EMBED, d1), lambda i: (0, 0, 0)),
            pl.BlockSpec((1, d1), lambda i: (0, 0)),
            pl.BlockSpec((d1, d2), lambda i: (0, 0)),
            pl.BlockSpec((1, d2), lambda i: (0, 0)),
            pl.BlockSpec((d2, d3), lambda i: (0, 0)),
            pl.BlockSpec((1, d3), lambda i: (0, 0)),
        ],
        out_specs=pl.BlockSpec((_TILE, d3), lambda i: (i, 0)),
        out_shape=jax.ShapeDtypeStruct((_B, d3), jnp.float32),
        scratch_shapes=[
            pltpu.VMEM((_K, d1), jnp.bfloat16),
            pltpu.VMEM((_K, d1), jnp.bfloat16),
            pltpu.VMEM((d1, d2), jnp.bfloat16),
            pltpu.VMEM((d2, d3), jnp.bfloat16),
        ],
    )(vals, jnp.asarray(np.vstack([_SEL] * 3)).astype(jnp.bfloat16),
      jnp.asarray(_LO), jnp.asarray(_HI), tables, w0r,
      b0.reshape(1, d1), W1, b1.reshape(1, d2), W2, b2.reshape(1, d3))
    return out


# speedup vs baseline: 33.6669x; 1.1137x over previous
import numpy as np
import jax
import jax.numpy as jnp
from jax.experimental import pallas as pl
from jax.experimental.pallas import tpu as pltpu

_B = 16384
_EMBED = 128
_NB = 21
_SCALES = (20.0, 2000.0, 2000.0, 2000.0, 2000.0, 1000.0, 40.0, 244.0)
_F = len(_SCALES)
_K = _F * _NB
_TILE = 4096


def _build_consts():
    sel = np.zeros((_F, _K), np.float32)
    lo = np.full((1, _K), -np.inf, np.float32)
    hi = np.full((1, _K), np.inf, np.float32)
    for f, scale in enumerate(_SCALES):
        bnds = np.linspace(0.0, float(scale), 20).astype(np.float32)
        sel[f, f * _NB:(f + 1) * _NB] = 1.0
        lo[0, f * _NB + 1:(f + 1) * _NB] = bnds
        hi[0, f * _NB:(f + 1) * _NB - 1] = bnds
    return sel, lo, hi


_SEL, _LO, _HI = _build_consts()


def _body(vals_ref, sel_ref, lo_ref, hi_ref, tables_ref, w0_ref, b0_ref,
          w1_ref, b1_ref, w2_ref, b2_ref, out_ref,
          mhi_ref, mlo_ref, w1b_ref, w2b_ref):
    @pl.when(pl.program_id(0) == 0)
    def _init():
        for f in range(_F):
            mf = jnp.dot(tables_ref[f], w0_ref[f],
                         preferred_element_type=jnp.float32,
                         precision=jax.lax.Precision.HIGHEST)
            mh = mf.astype(jnp.bfloat16)
            mhi_ref[f * _NB:(f + 1) * _NB, :] = mh
            mlo_ref[f * _NB:(f + 1) * _NB, :] = (
                mf - mh.astype(jnp.float32)).astype(jnp.bfloat16)
        w1b_ref[...] = w1_ref[...].astype(jnp.bfloat16)
        w2b_ref[...] = w2_ref[...].astype(jnp.bfloat16)

    v = vals_ref[...]
    v1 = v.astype(jnp.bfloat16)
    r = v - v1.astype(jnp.float32)
    v2 = r.astype(jnp.bfloat16)
    v3 = (r - v2.astype(jnp.float32)).astype(jnp.bfloat16)
    v123 = jnp.concatenate([v1, v2, v3], axis=1)
    x = jnp.dot(v123, sel_ref[...], preferred_element_type=jnp.float32)
    oh = jnp.logical_and(x >= lo_ref[...], x < hi_ref[...]).astype(jnp.bfloat16)
    h = jnp.dot(oh, mhi_ref[...], preferred_element_type=jnp.float32)
    h = jnp.maximum(h + b0_ref[...], 0.0)
    h = jnp.dot(h.astype(jnp.bfloat16), w1b_ref[...],
                preferred_element_type=jnp.float32)
    h = jnp.maximum(h + b1_ref[...], 0.0)
    out_ref[...] = (jnp.dot(h.astype(jnp.bfloat16), w2b_ref[...],
                            preferred_element_type=jnp.float32)
                    + b2_ref[...])


def kernel(addr_number, addr_number_table, login_num_30d, login_num_30d_table,
           last7d_login_num, last7d_login_num_table, share_num_360d,
           share_num_360d_table, gmv_30d, gmv_30d_table, gmv_7d, gmv_7d_table,
           orders_30d, orders_30d_table, orders_7d, orders_7d_table,
           W0, b0, W1, b1, W2, b2):
    tables = jnp.stack([addr_number_table, login_num_30d_table,
                        last7d_login_num_table, share_num_360d_table,
                        gmv_30d_table, gmv_7d_table, orders_30d_table,
                        orders_7d_table])
    d1 = W0.shape[1]
    w0r = W0.reshape(_F, _EMBED, d1)
    vals = jnp.stack([addr_number, login_num_30d, last7d_login_num,
                      share_num_360d, gmv_30d, gmv_7d, orders_30d, orders_7d],
                     axis=1)

    d2, d3 = W1.shape[1], W2.shape[1]
    grid = _B // _TILE
    out = pl.pallas_call(
        _body,
        grid=(grid,),
        in_specs=[
            pl.BlockSpec((_TILE, _F), lambda i: (i, 0)),
            pl.BlockSpec((3 * _F, _K), lambda i: (0, 0)),
            pl.BlockSpec((1, _K), lambda i: (0, 0)),
            pl.BlockSpec((1, _K), lambda i: (0, 0)),
            pl.BlockSpec((_F, _NB, _EMBED), lambda i: (0, 0, 0)),
            pl.BlockSpec((_F, _EMBED, d1), lambda i: (0, 0, 0)),
            pl.BlockSpec((1, d1), lambda i: (0, 0)),
            pl.BlockSpec((d1, d2), lambda i: (0, 0)),
            pl.BlockSpec((1, d2), lambda i: (0, 0)),
            pl.BlockSpec((d2, d3), lambda i: (0, 0)),
            pl.BlockSpec((1, d3), lambda i: (0, 0)),
        ],
        out_specs=pl.BlockSpec((_TILE, d3), lambda i: (i, 0)),
        out_shape=jax.ShapeDtypeStruct((_B, d3), jnp.float32),
        scratch_shapes=[
            pltpu.VMEM((_K, d1), jnp.bfloat16),
            pltpu.VMEM((_K, d1), jnp.bfloat16),
            pltpu.VMEM((d1, d2), jnp.bfloat16),
            pltpu.VMEM((d2, d3), jnp.bfloat16),
        ],
    )(vals, jnp.asarray(np.vstack([_SEL] * 3)).astype(jnp.bfloat16),
      jnp.asarray(_LO), jnp.asarray(_HI), tables, w0r,
      b0.reshape(1, d1), W1, b1.reshape(1, d2), W2, b2.reshape(1, d3))
    return out
